# baseline (speedup 1.0000x reference)
import jax
import jax.numpy as jnp
from jax.experimental import pallas as pl
from jax.experimental.pallas import tpu as pltpu

N = 320000
D = 128
H = 64
S = 10000
C = 2560
G = N // C
W = 128


def _attn_kernel(x_ref, bi_ref, w_ref, v_ref, out_ref, acc_ref, den_ref):
    c = pl.program_id(0)

    @pl.when(c == 0)
    def _init():
        acc_ref[...] = jnp.zeros_like(acc_ref)
        den_ref[...] = jnp.zeros_like(den_ref)

    U = C // 2
    row = jax.lax.broadcasted_iota(jnp.int32, (W, U), 0)

    for j in range(2):
        x = x_ref[pl.ds(j * U, U), :]
        h = jnp.tanh(jax.lax.dot(x, w_ref[...]))
        beta = jax.lax.dot(h, v_ref[...])
        e = jnp.exp(beta)
        wgt = (x * e).astype(jnp.bfloat16)
        e_bf = e.astype(jnp.bfloat16)

        ids = bi_ref[0, :, pl.ds(j * U, U)]
        base = (jnp.min(ids) // 8) * 8
        local = ids - base
        nwin = jnp.max(local) // W + 1

        def window(k, row=row, local=local, base=base, wgt=wgt, e_bf=e_bf):
            oht = (row + k * W == local).astype(jnp.bfloat16)
            win_num = jax.lax.dot(oht, wgt,
                                  preferred_element_type=jnp.float32)
            win_den = jax.lax.dot(oht, e_bf,
                                  preferred_element_type=jnp.float32)
            b = base + k * W
            acc_ref[pl.ds(b, W), :] += win_num
            den_ref[pl.ds(b, W), :] += win_den

        window(0)

        @pl.when(nwin > 1)
        def _extra(window=window, nwin=nwin):
            jax.lax.fori_loop(1, nwin, lambda k, c: (window(k), c)[1], 0)

    @pl.when(c == G - 1)
    def _fin():
        num = acc_ref[pl.ds(0, S), :]
        den = den_ref[pl.ds(0, S), :]
        out_ref[...] = jnp.where(den > 0, num / den, 0.0)


def kernel(embed, batch_index, W_a, V_a):
    bi3 = batch_index.reshape(G, 1, C)
    return pl.pallas_call(
        _attn_kernel,
        grid=(G,),
        in_specs=[
            pl.BlockSpec((C, D), lambda c: (c, 0)),
            pl.BlockSpec((1, 1, C), lambda c: (c, 0, 0)),
            pl.BlockSpec((D, H), lambda c: (0, 0)),
            pl.BlockSpec((H, 1), lambda c: (0, 0)),
        ],
        out_specs=pl.BlockSpec((S, D), lambda c: (0, 0)),
        out_shape=jax.ShapeDtypeStruct((S, D), jnp.float32),
        scratch_shapes=[
            pltpu.VMEM((S + W, D), jnp.float32),
            pltpu.VMEM((S + W, 1), jnp.float32),
        ],
    )(embed, bi3, W_a, V_a)


# speedup vs baseline: 1.3999x; 1.3999x over previous
import jax
import jax.numpy as jnp
from jax.experimental import pallas as pl
from jax.experimental.pallas import tpu as pltpu

N = 320000
D = 128
H = 64
S = 10000
C = 3200
G = N // C
W = 128


def _attn_kernel(x_ref, bi_ref, w_ref, v_ref, out_ref, acc_ref, den_ref):
    c = pl.program_id(0)

    @pl.when(c == 0)
    def _init():
        acc_ref[...] = jnp.zeros_like(acc_ref)
        den_ref[...] = jnp.zeros_like(den_ref)

    x = x_ref[...]
    h = jnp.tanh(jax.lax.dot(x, w_ref[...]))
    beta = jax.lax.dot(h, v_ref[...])
    e = jnp.exp(beta)
    wgt = (x * e).astype(jnp.bfloat16)
    e_bf = e.astype(jnp.bfloat16)

    ids = bi_ref[0]
    base = (jnp.min(ids) // 8) * 8
    local = ids - base
    nwin = jnp.max(local) // W + 1

    row = jax.lax.broadcasted_iota(jnp.int32, (W, C), 0)

    def window(k):
        oht = (row + k * W == local).astype(jnp.bfloat16)
        win_num = jax.lax.dot(oht, wgt,
                              preferred_element_type=jnp.float32)
        win_den = jax.lax.dot(oht, e_bf,
                              preferred_element_type=jnp.float32)
        b = base + k * W
        acc_ref[pl.ds(b, W), :] += win_num
        den_ref[pl.ds(b, W), :] += win_den

    window(0)

    @pl.when(nwin > 1)
    def _extra():
        jax.lax.fori_loop(1, nwin, lambda k, c: (window(k), c)[1], 0)

    @pl.when(c == G - 1)
    def _fin():
        num = acc_ref[pl.ds(0, S), :]
        den = den_ref[pl.ds(0, S), :]
        out_ref[...] = jnp.where(den > 0, num / den, 0.0)


def kernel(embed, batch_index, W_a, V_a):
    bi3 = batch_index.reshape(G, 1, C)
    return pl.pallas_call(
        _attn_kernel,
        grid=(G,),
        in_specs=[
            pl.BlockSpec((C, D), lambda c: (c, 0)),
            pl.BlockSpec((1, 1, C), lambda c: (c, 0, 0)),
            pl.BlockSpec((D, H), lambda c: (0, 0)),
            pl.BlockSpec((H, 1), lambda c: (0, 0)),
        ],
        out_specs=pl.BlockSpec((S, D), lambda c: (0, 0)),
        out_shape=jax.ShapeDtypeStruct((S, D), jnp.float32),
        scratch_shapes=[
            pltpu.VMEM((S + W, D), jnp.float32),
            pltpu.VMEM((S + W, 1), jnp.float32),
        ],
    )(embed, bi3, W_a, V_a)
